# baseline (device time: 60309 ns/iter reference)
import jax
import jax.numpy as jnp
from jax import lax
from jax.experimental import pallas as pl
from jax.experimental.pallas import tpu as pltpu

N_DEV = 32
F8 = jnp.float8_e4m3fn
NBUF = 4
NCHUNK = 4


def kernel(x, w_mat, scale_x, scale_w):
    m_per, k = x.shape
    _, n = w_mat.shape
    n_per = n // N_DEV
    out_m = N_DEV * m_per
    rpc = k // NCHUNK

    def body(x_ref, w_hbm, sx_ref, sw_ref, out_ref,
             x8_ref, w_ring, send_buf, recv_buf,
             dma_sems, send_sems, recv_sems):
        g = pl.program_id(0)
        my = lax.axis_index("i")
        t = lax.rem(my + g, N_DEV)

        def issue_tile(step):
            tt = lax.rem(my + step, N_DEV)
            slot = lax.rem(step, NBUF)
            for c in range(NCHUNK):
                pltpu.make_async_copy(
                    w_hbm.at[pl.ds(c * rpc, rpc), pl.ds(tt * n_per, n_per)],
                    w_ring.at[slot, pl.ds(c * rpc, rpc), :],
                    dma_sems.at[slot, c],
                ).start()

        def wait_tile(step):
            slot = lax.rem(step, NBUF)
            for c in range(NCHUNK):
                pltpu.make_async_copy(
                    w_hbm.at[pl.ds(c * rpc, rpc), pl.ds(0, n_per)],
                    w_ring.at[slot, pl.ds(c * rpc, rpc), :],
                    dma_sems.at[slot, c],
                ).wait()

        @pl.when(g == 0)
        def _():
            x8_ref[...] = x_ref[...].astype(jnp.bfloat16)
            for s in range(NBUF):
                issue_tile(jnp.int32(s))

        wait_tile(g)
        slot = lax.rem(g, NBUF)
        kh = k // 2
        dn = (((1,), (0,)), ((), ()))
        w8a = w_ring[slot, pl.ds(0, kh), :].astype(jnp.bfloat16)
        acc = lax.dot_general(
            x8_ref[:, pl.ds(0, kh)], w8a, dn,
            preferred_element_type=jnp.float32,
        )
        w8b = w_ring[slot, pl.ds(kh, kh), :].astype(jnp.bfloat16)
        acc = acc + lax.dot_general(
            x8_ref[:, pl.ds(kh, kh)], w8b, dn,
            preferred_element_type=jnp.float32,
        )

        @pl.when(g < N_DEV - NBUF)
        def _():
            issue_tile(g + NBUF)

        yv = acc * (sx_ref[0] * sw_ref[0])
        yv = yv * (1.0 / (1.0 + jnp.exp(-jnp.clip(yv, -60.0, 60.0))))
        yb = yv.astype(jnp.bfloat16)

        @pl.when(g == 0)
        def _():
            recv_buf[my] = yb

        @pl.when(g > 0)
        def _():
            send_buf[g] = yb
            rdma = pltpu.make_async_remote_copy(
                src_ref=send_buf.at[g],
                dst_ref=recv_buf.at[my],
                send_sem=send_sems.at[g],
                recv_sem=recv_sems.at[my],
                device_id=(t,),
                device_id_type=pl.DeviceIdType.MESH,
            )
            rdma.start()

        @pl.when(g == N_DEV - 1)
        def _():
            for s in range(N_DEV):
                @pl.when(s != my)
                def _(s=s):
                    recv = pltpu.make_async_remote_copy(
                        src_ref=send_buf.at[1],
                        dst_ref=recv_buf.at[s],
                        send_sem=send_sems.at[1],
                        recv_sem=recv_sems.at[s],
                        device_id=(my,),
                        device_id_type=pl.DeviceIdType.MESH,
                    )
                    recv.wait_recv()
            out_ref[...] = recv_buf[...].reshape(out_m, n_per).astype(jnp.float32)
            for d in range(1, N_DEV):
                snd = pltpu.make_async_remote_copy(
                    src_ref=send_buf.at[d],
                    dst_ref=recv_buf.at[my],
                    send_sem=send_sems.at[d],
                    recv_sem=recv_sems.at[my],
                    device_id=(my,),
                    device_id_type=pl.DeviceIdType.MESH,
                )
                snd.wait_send()

    grid = (N_DEV,)
    return pl.pallas_call(
        body,
        grid=grid,
        out_shape=jax.ShapeDtypeStruct((out_m, n_per), jnp.float32),
        in_specs=[
            pl.BlockSpec((m_per, k), lambda g: (0, 0)),
            pl.BlockSpec(memory_space=pl.ANY),
            pl.BlockSpec(memory_space=pltpu.SMEM),
            pl.BlockSpec(memory_space=pltpu.SMEM),
        ],
        out_specs=pl.BlockSpec((out_m, n_per), lambda g: (0, 0)),
        scratch_shapes=[
            pltpu.VMEM((m_per, k), jnp.bfloat16),
            pltpu.VMEM((NBUF, k, n_per), jnp.float32),
            pltpu.VMEM((N_DEV, m_per, n_per), jnp.bfloat16),
            pltpu.VMEM((N_DEV, m_per, n_per), jnp.bfloat16),
            pltpu.SemaphoreType.DMA((NBUF, NCHUNK)),
            pltpu.SemaphoreType.DMA((N_DEV,)),
            pltpu.SemaphoreType.DMA((N_DEV,)),
        ],
        compiler_params=pltpu.CompilerParams(
            dimension_semantics=("arbitrary",),
        ),
    )(x, w_mat, scale_x, scale_w)


# device time: 60086 ns/iter; 1.0037x vs baseline; 1.0037x over previous
import jax
import jax.numpy as jnp
from jax import lax
from jax.experimental import pallas as pl
from jax.experimental.pallas import tpu as pltpu

N_DEV = 32
F8 = jnp.float8_e4m3fn
NBUF = 4
NCHUNK = 4


def kernel(x, w_mat, scale_x, scale_w):
    m_per, k = x.shape
    _, n = w_mat.shape
    n_per = n // N_DEV
    out_m = N_DEV * m_per
    rpc = k // NCHUNK

    def body(x_ref, w_hbm, sx_ref, sw_ref, out_ref,
             x8_ref, w_ring, send_buf, recv_buf,
             dma_sems, send_sems, recv_sems):
        g = pl.program_id(0)
        my = lax.axis_index("i")
        t = lax.rem(my + g, N_DEV)

        def issue_tile(step):
            tt = lax.rem(my + step, N_DEV)
            slot = lax.rem(step, NBUF)
            for c in range(NCHUNK):
                pltpu.make_async_copy(
                    w_hbm.at[pl.ds(c * rpc, rpc), pl.ds(tt * n_per, n_per)],
                    w_ring.at[slot, pl.ds(c * rpc, rpc), :],
                    dma_sems.at[slot, c],
                ).start()

        def wait_tile(step):
            slot = lax.rem(step, NBUF)
            for c in range(NCHUNK):
                pltpu.make_async_copy(
                    w_hbm.at[pl.ds(c * rpc, rpc), pl.ds(0, n_per)],
                    w_ring.at[slot, pl.ds(c * rpc, rpc), :],
                    dma_sems.at[slot, c],
                ).wait()

        @pl.when(g == 0)
        def _():
            x8_ref[...] = x_ref[...].astype(F8)
            for s in range(NBUF):
                issue_tile(jnp.int32(s))

        wait_tile(g)
        slot = lax.rem(g, NBUF)
        kh = k // 2
        dn = (((1,), (0,)), ((), ()))
        w8a = w_ring[slot, pl.ds(0, kh), :].astype(F8)
        acc = lax.dot_general(
            x8_ref[:, pl.ds(0, kh)], w8a, dn,
            preferred_element_type=jnp.float32,
        )
        w8b = w_ring[slot, pl.ds(kh, kh), :].astype(F8)
        acc = acc + lax.dot_general(
            x8_ref[:, pl.ds(kh, kh)], w8b, dn,
            preferred_element_type=jnp.float32,
        )

        @pl.when(g < N_DEV - NBUF)
        def _():
            issue_tile(g + NBUF)

        yv = acc * (sx_ref[0] * sw_ref[0])
        yv = yv * (1.0 / (1.0 + jnp.exp(-jnp.clip(yv, -60.0, 60.0))))
        yb = yv.astype(jnp.bfloat16)

        @pl.when(g == 0)
        def _():
            recv_buf[my] = yb

        @pl.when(g > 0)
        def _():
            send_buf[g] = yb
            rdma = pltpu.make_async_remote_copy(
                src_ref=send_buf.at[g],
                dst_ref=recv_buf.at[my],
                send_sem=send_sems.at[g],
                recv_sem=recv_sems.at[my],
                device_id=(t,),
                device_id_type=pl.DeviceIdType.MESH,
            )
            rdma.start()

        @pl.when(g == N_DEV - 1)
        def _():
            for s in range(N_DEV):
                @pl.when(s != my)
                def _(s=s):
                    recv = pltpu.make_async_remote_copy(
                        src_ref=send_buf.at[1],
                        dst_ref=recv_buf.at[s],
                        send_sem=send_sems.at[1],
                        recv_sem=recv_sems.at[s],
                        device_id=(my,),
                        device_id_type=pl.DeviceIdType.MESH,
                    )
                    recv.wait_recv()
            out_ref[...] = recv_buf[...].reshape(out_m, n_per).astype(jnp.float32)
            for d in range(1, N_DEV):
                snd = pltpu.make_async_remote_copy(
                    src_ref=send_buf.at[d],
                    dst_ref=recv_buf.at[my],
                    send_sem=send_sems.at[d],
                    recv_sem=recv_sems.at[my],
                    device_id=(my,),
                    device_id_type=pl.DeviceIdType.MESH,
                )
                snd.wait_send()

    grid = (N_DEV,)
    return pl.pallas_call(
        body,
        grid=grid,
        out_shape=jax.ShapeDtypeStruct((out_m, n_per), jnp.float32),
        in_specs=[
            pl.BlockSpec((m_per, k), lambda g: (0, 0)),
            pl.BlockSpec(memory_space=pl.ANY),
            pl.BlockSpec(memory_space=pltpu.SMEM),
            pl.BlockSpec(memory_space=pltpu.SMEM),
        ],
        out_specs=pl.BlockSpec((out_m, n_per), lambda g: (0, 0)),
        scratch_shapes=[
            pltpu.VMEM((m_per, k), F8),
            pltpu.VMEM((NBUF, k, n_per), jnp.float32),
            pltpu.VMEM((N_DEV, m_per, n_per), jnp.bfloat16),
            pltpu.VMEM((N_DEV, m_per, n_per), jnp.bfloat16),
            pltpu.SemaphoreType.DMA((NBUF, NCHUNK)),
            pltpu.SemaphoreType.DMA((N_DEV,)),
            pltpu.SemaphoreType.DMA((N_DEV,)),
        ],
        compiler_params=pltpu.CompilerParams(
            dimension_semantics=("arbitrary",),
        ),
    )(x, w_mat, scale_x, scale_w)
